# Initial kernel scaffold; baseline (speedup 1.0000x reference)
import jax
import jax.numpy as jnp
from jax.experimental import pallas as pl


def kernel(view, edge_index, W1, b1, W2, b2):
    raise NotImplementedError("write your pallas kernel here")


# speedup vs baseline: 21.5611x; 21.5611x over previous
import functools

import jax
import jax.numpy as jnp
from jax import lax
from jax.experimental import pallas as pl
from jax.experimental.pallas import tpu as pltpu
from jax.experimental.pallas import tpu_sc as plsc

N_NODES = 10000
N_EDGES = 320000
D_IN = 128
D_HID = 64
N_ACT = 16

NP = 10240
NC, NS = 2, 16
NW = NC * NS
CHUNK = 128
NCH = (N_EDGES + NW * CHUNK - 1) // (NW * CHUNK)
EPW = NCH * CHUNK
EP = EPW * NW
RPS = NP // NS


def _make_sc_agg(d):
    mesh = plsc.VectorSubcoreMesh(core_axis_name="c", subcore_axis_name="s")

    def body(g_hbm, src_hbm, dst_hbm, zeros_hbm, out_hbm, src_v, dst_v, buf, acc):
        cid = lax.axis_index("c")
        sid = lax.axis_index("s")
        wid = cid * NS + sid
        pltpu.sync_copy(src_hbm.at[wid], src_v)
        pltpu.sync_copy(dst_hbm.at[wid], dst_v)
        sl = pl.ds(sid * RPS, RPS)
        pltpu.sync_copy(zeros_hbm.at[sl], acc.at[sl])
        plsc.subcore_barrier()

        def step(j, carry):
            pltpu.sync_copy(g_hbm.at[src_v.at[j]], buf)
            pltpu.sync_copy(buf, acc.at[dst_v.at[j]], add=True)
            return carry

        lax.fori_loop(0, NCH, step, 0)
        plsc.subcore_barrier()
        pltpu.sync_copy(acc.at[sl], out_hbm.at[cid, sl])

    return pl.kernel(
        body,
        out_type=jax.ShapeDtypeStruct((NC, NP, d), jnp.float32),
        mesh=mesh,
        compiler_params=pltpu.CompilerParams(use_tc_tiling_on_sc=False),
        scratch_types=[
            pltpu.VMEM((NCH, CHUNK), jnp.int32),
            pltpu.VMEM((NCH, CHUNK), jnp.int32),
            pltpu.VMEM((CHUNK, d), jnp.float32),
            pltpu.VMEM_SHARED((NP, d), jnp.float32),
        ],
    )


BM = 1024


def _tc_a_body(view_ref, w1t_ref, parts_ref, g1_ref, dis_ref):
    deg = parts_ref[0, :, 0:1] + parts_ref[1, :, 0:1] + 1.0
    dis = lax.rsqrt(deg)
    h1 = jnp.dot(view_ref[...], w1t_ref[...], preferred_element_type=jnp.float32)
    g1_ref[...] = h1 * dis
    dis_ref[...] = dis


def _tc_b_body(parts_ref, g1_ref, dis_ref, b1_ref, w2t_ref, g2_ref):
    dis = dis_ref[...]
    s = parts_ref[0] + parts_ref[1] + g1_ref[...]
    x = jnp.maximum(s * dis + b1_ref[...], 0.0)
    h2 = jnp.dot(x, w2t_ref[...], preferred_element_type=jnp.float32)
    g2_ref[...] = h2 * dis


def _tc_c_body(parts_ref, g2_ref, dis_ref, b2_ref, out_ref):
    s = parts_ref[0] + parts_ref[1] + g2_ref[...]
    out_ref[...] = s * dis_ref[...] + b2_ref[...]


def _row_spec(d):
    return pl.BlockSpec((BM, d), lambda i: (i, 0))


def _parts_spec(d):
    return pl.BlockSpec((NC, BM, d), lambda i: (0, i, 0))


def _full_spec(a, b):
    return pl.BlockSpec((a, b), lambda i: (0, 0))


_GRID = (NP // BM,)

_tc_a = pl.pallas_call(
    _tc_a_body,
    grid=_GRID,
    in_specs=[_row_spec(D_IN), _full_spec(D_IN, D_HID), _parts_spec(8)],
    out_specs=[_row_spec(D_HID), _row_spec(1)],
    out_shape=[
        jax.ShapeDtypeStruct((NP, D_HID), jnp.float32),
        jax.ShapeDtypeStruct((NP, 1), jnp.float32),
    ],
)

_tc_b = pl.pallas_call(
    _tc_b_body,
    grid=_GRID,
    in_specs=[_parts_spec(D_HID), _row_spec(D_HID), _row_spec(1),
              _full_spec(1, D_HID), _full_spec(D_HID, N_ACT)],
    out_specs=_row_spec(N_ACT),
    out_shape=jax.ShapeDtypeStruct((NP, N_ACT), jnp.float32),
)

_tc_c = pl.pallas_call(
    _tc_c_body,
    grid=_GRID,
    in_specs=[_parts_spec(N_ACT), _row_spec(N_ACT), _row_spec(1),
              _full_spec(1, N_ACT)],
    out_specs=_row_spec(N_ACT),
    out_shape=jax.ShapeDtypeStruct((NP, N_ACT), jnp.float32),
)

_agg_deg = _make_sc_agg(8)
_agg_h = _make_sc_agg(D_HID)
_agg_o = _make_sc_agg(N_ACT)


def kernel(view, edge_index, W1, b1, W2, b2):
    src = edge_index[0].astype(jnp.int32)
    dst = edge_index[1].astype(jnp.int32)
    pad = EP - N_EDGES
    fill = jnp.full((pad,), N_NODES, jnp.int32)
    src3 = jnp.concatenate([src, fill]).reshape(NW, NCH, CHUNK)
    dst3 = jnp.concatenate([dst, fill]).reshape(NW, NCH, CHUNK)

    view_p = jnp.pad(view, ((0, NP - N_NODES), (0, 0)))
    ones8 = jnp.ones((NP, 8), jnp.float32)
    z8 = jnp.zeros((NP, 8), jnp.float32)
    z64 = jnp.zeros((NP, D_HID), jnp.float32)
    z16 = jnp.zeros((NP, N_ACT), jnp.float32)

    deg_parts = _agg_deg(ones8, src3, dst3, z8)
    g1, dis = _tc_a(view_p, W1.T, deg_parts)
    parts1 = _agg_h(g1, src3, dst3, z64)
    g2 = _tc_b(parts1, g1, dis, b1.reshape(1, D_HID), W2.T)
    parts2 = _agg_o(g2, src3, dst3, z16)
    out = _tc_c(parts2, g2, dis, b2.reshape(1, N_ACT))
    return out[:N_NODES]


# speedup vs baseline: 24.6694x; 1.1442x over previous
import functools

import jax
import jax.numpy as jnp
from jax import lax
from jax.experimental import pallas as pl
from jax.experimental.pallas import tpu as pltpu
from jax.experimental.pallas import tpu_sc as plsc

N_NODES = 10000
N_EDGES = 320000
D_IN = 128
D_HID = 64
N_ACT = 16

NP = 10240
NC, NS = 2, 16
NW = NC * NS
CHUNK = 128
K = 4
NG = 20
NCH = NG * K
EPW = NCH * CHUNK
EP = EPW * NW
RPS = NP // NS


def _make_sc_agg(d):
    mesh = plsc.VectorSubcoreMesh(core_axis_name="c", subcore_axis_name="s")

    def body(g_hbm, src_hbm, dst_hbm, zeros_hbm, out_hbm,
             src_v, dst_v, buf, acc, semg, sems):
        cid = lax.axis_index("c")
        sid = lax.axis_index("s")
        wid = cid * NS + sid
        pltpu.sync_copy(src_hbm.at[wid], src_v)
        pltpu.sync_copy(dst_hbm.at[wid], dst_v)
        sl = pl.ds(sid * RPS, RPS)
        pltpu.sync_copy(zeros_hbm.at[sl], acc.at[sl])
        plsc.subcore_barrier()

        for j in range(K):
            pltpu.async_copy(g_hbm.at[src_v.at[j]], buf.at[0, j], semg)

        def group(i, carry):
            pb = lax.rem(i, 2)
            nb = 1 - pb
            base = i * K
            nbase = base + K

            @pl.when(i + 1 < NG)
            def _fire_next():
                for j in range(K):
                    pltpu.async_copy(g_hbm.at[src_v.at[nbase + j]],
                                     buf.at[nb, j], semg)

            for j in range(K):
                pltpu.make_async_copy(g_hbm.at[src_v.at[base + j]],
                                      buf.at[pb, j], semg).wait()
                pltpu.async_copy(buf.at[pb, j], acc.at[dst_v.at[base + j]],
                                 sems, add=True)
            for j in range(K):
                pltpu.make_async_copy(buf.at[pb, j],
                                      acc.at[dst_v.at[base + j]], sems).wait()
            return carry

        lax.fori_loop(0, NG, group, 0)
        plsc.subcore_barrier()
        pltpu.sync_copy(acc.at[sl], out_hbm.at[cid, sl])

    return pl.kernel(
        body,
        out_type=jax.ShapeDtypeStruct((NC, NP, d), jnp.float32),
        mesh=mesh,
        compiler_params=pltpu.CompilerParams(use_tc_tiling_on_sc=False),
        scratch_types=[
            pltpu.VMEM((NCH, CHUNK), jnp.int32),
            pltpu.VMEM((NCH, CHUNK), jnp.int32),
            pltpu.VMEM((2, K, CHUNK, d), jnp.float32),
            pltpu.VMEM_SHARED((NP, d), jnp.float32),
            pltpu.SemaphoreType.DMA,
            pltpu.SemaphoreType.DMA,
        ],
    )


def _make_sc_deg():
    mesh = plsc.VectorSubcoreMesh(core_axis_name="c", subcore_axis_name="s")
    d = 8
    KD = 10

    def body(ones_hbm, dst_hbm, zeros_hbm, out_hbm, dst_v, ones_v, acc, sems):
        cid = lax.axis_index("c")
        sid = lax.axis_index("s")
        wid = cid * NS + sid
        pltpu.sync_copy(dst_hbm.at[wid], dst_v)
        pltpu.sync_copy(ones_hbm.at[pl.ds(0, CHUNK)], ones_v)
        sl = pl.ds(sid * RPS, RPS)
        pltpu.sync_copy(zeros_hbm.at[sl], acc.at[sl])
        plsc.subcore_barrier()

        def group(i, carry):
            base = i * KD
            for j in range(KD):
                pltpu.async_copy(ones_v, acc.at[dst_v.at[base + j]],
                                 sems, add=True)
            for j in range(KD):
                pltpu.make_async_copy(ones_v,
                                      acc.at[dst_v.at[base + j]], sems).wait()
            return carry

        lax.fori_loop(0, NCH // KD, group, 0)
        plsc.subcore_barrier()
        pltpu.sync_copy(acc.at[sl], out_hbm.at[cid, sl])

    return pl.kernel(
        body,
        out_type=jax.ShapeDtypeStruct((NC, NP, d), jnp.float32),
        mesh=mesh,
        compiler_params=pltpu.CompilerParams(use_tc_tiling_on_sc=False),
        scratch_types=[
            pltpu.VMEM((NCH, CHUNK), jnp.int32),
            pltpu.VMEM((CHUNK, d), jnp.float32),
            pltpu.VMEM_SHARED((NP, d), jnp.float32),
            pltpu.SemaphoreType.DMA,
        ],
    )


BM = 1024


def _tc_a_body(view_ref, w1t_ref, parts_ref, g1_ref, dis_ref):
    deg = parts_ref[0, :, 0:1] + parts_ref[1, :, 0:1] + 1.0
    dis = lax.rsqrt(deg)
    h1 = jnp.dot(view_ref[...], w1t_ref[...], preferred_element_type=jnp.float32)
    g1_ref[...] = h1 * dis
    dis_ref[...] = dis


def _tc_b_body(parts_ref, g1_ref, dis_ref, b1_ref, w2t_ref, g2_ref):
    dis = dis_ref[...]
    s = parts_ref[0] + parts_ref[1] + g1_ref[...]
    x = jnp.maximum(s * dis + b1_ref[...], 0.0)
    h2 = jnp.dot(x, w2t_ref[...], preferred_element_type=jnp.float32)
    g2_ref[...] = h2 * dis


def _tc_c_body(parts_ref, g2_ref, dis_ref, b2_ref, out_ref):
    s = parts_ref[0] + parts_ref[1] + g2_ref[...]
    out_ref[...] = s * dis_ref[...] + b2_ref[...]


def _row_spec(d):
    return pl.BlockSpec((BM, d), lambda i: (i, 0))


def _parts_spec(d):
    return pl.BlockSpec((NC, BM, d), lambda i: (0, i, 0))


def _full_spec(a, b):
    return pl.BlockSpec((a, b), lambda i: (0, 0))


_GRID = (NP // BM,)

_tc_a = pl.pallas_call(
    _tc_a_body,
    grid=_GRID,
    in_specs=[_row_spec(D_IN), _full_spec(D_IN, D_HID), _parts_spec(8)],
    out_specs=[_row_spec(D_HID), _row_spec(1)],
    out_shape=[
        jax.ShapeDtypeStruct((NP, D_HID), jnp.float32),
        jax.ShapeDtypeStruct((NP, 1), jnp.float32),
    ],
)

_tc_b = pl.pallas_call(
    _tc_b_body,
    grid=_GRID,
    in_specs=[_parts_spec(D_HID), _row_spec(D_HID), _row_spec(1),
              _full_spec(1, D_HID), _full_spec(D_HID, N_ACT)],
    out_specs=_row_spec(N_ACT),
    out_shape=jax.ShapeDtypeStruct((NP, N_ACT), jnp.float32),
)

_tc_c = pl.pallas_call(
    _tc_c_body,
    grid=_GRID,
    in_specs=[_parts_spec(N_ACT), _row_spec(N_ACT), _row_spec(1),
              _full_spec(1, N_ACT)],
    out_specs=_row_spec(N_ACT),
    out_shape=jax.ShapeDtypeStruct((NP, N_ACT), jnp.float32),
)

_agg_deg = _make_sc_deg()
_agg_h = _make_sc_agg(D_HID)
_agg_o = _make_sc_agg(N_ACT)


def kernel(view, edge_index, W1, b1, W2, b2):
    src = edge_index[0].astype(jnp.int32)
    dst = edge_index[1].astype(jnp.int32)
    pad = EP - N_EDGES
    fill = jnp.full((pad,), N_NODES, jnp.int32)
    src3 = jnp.concatenate([src, fill]).reshape(NW, NCH, CHUNK)
    dst3 = jnp.concatenate([dst, fill]).reshape(NW, NCH, CHUNK)

    view_p = jnp.pad(view, ((0, NP - N_NODES), (0, 0)))
    ones8 = jnp.ones((NP, 8), jnp.float32)
    z8 = jnp.zeros((NP, 8), jnp.float32)
    z64 = jnp.zeros((NP, D_HID), jnp.float32)
    z16 = jnp.zeros((NP, N_ACT), jnp.float32)

    deg_parts = _agg_deg(ones8, dst3, z8)
    g1, dis = _tc_a(view_p, W1.T, deg_parts)
    parts1 = _agg_h(g1, src3, dst3, z64)
    g2 = _tc_b(parts1, g1, dis, b1.reshape(1, D_HID), W2.T)
    parts2 = _agg_o(g2, src3, dst3, z16)
    out = _tc_c(parts2, g2, dis, b2.reshape(1, N_ACT))
    return out[:N_NODES]


# speedup vs baseline: 28.1235x; 1.1400x over previous
import functools

import jax
import jax.numpy as jnp
from jax import lax
from jax.experimental import pallas as pl
from jax.experimental.pallas import tpu as pltpu
from jax.experimental.pallas import tpu_sc as plsc

N_NODES = 10000
N_EDGES = 320000
D_IN = 128
D_HID = 64
N_ACT = 16

NP = 10240
NC, NS = 2, 16
NW = NC * NS
CHUNK = 128
K = 4
NG = 20
NCH = NG * K
EPW = NCH * CHUNK
EP = EPW * NW
RPS = NP // NS


def _make_sc_agg(d):
    mesh = plsc.VectorSubcoreMesh(core_axis_name="c", subcore_axis_name="s")

    def body(g_hbm, src_hbm, dst_hbm, zeros_hbm, out_hbm,
             src_v, dst_v, buf, acc, semg, sems):
        cid = lax.axis_index("c")
        sid = lax.axis_index("s")
        wid = cid * NS + sid
        pltpu.sync_copy(src_hbm.at[wid], src_v)
        pltpu.sync_copy(dst_hbm.at[wid], dst_v)
        sl = pl.ds(sid * RPS, RPS)
        pltpu.sync_copy(zeros_hbm.at[sl], acc.at[sl])
        plsc.subcore_barrier()

        for j in range(K):
            pltpu.async_copy(g_hbm.at[src_v.at[j]], buf.at[0, j], semg)

        def group(i, carry):
            pb = lax.rem(i, 2)
            nb = 1 - pb
            base = i * K
            nbase = base + K

            @pl.when(i + 1 < NG)
            def _fire_next():
                for j in range(K):
                    pltpu.async_copy(g_hbm.at[src_v.at[nbase + j]],
                                     buf.at[nb, j], semg)

            for j in range(K):
                pltpu.make_async_copy(g_hbm.at[src_v.at[base + j]],
                                      buf.at[pb, j], semg).wait()
                pltpu.async_copy(buf.at[pb, j], acc.at[dst_v.at[base + j]],
                                 sems, add=True)
            for j in range(K):
                pltpu.make_async_copy(buf.at[pb, j],
                                      acc.at[dst_v.at[base + j]], sems).wait()
            return carry

        lax.fori_loop(0, NG, group, 0)
        plsc.subcore_barrier()
        pltpu.sync_copy(acc.at[sl], out_hbm.at[cid, sl])

    return pl.kernel(
        body,
        out_type=jax.ShapeDtypeStruct((NC, NP, d), jnp.float32),
        mesh=mesh,
        compiler_params=pltpu.CompilerParams(use_tc_tiling_on_sc=False),
        scratch_types=[
            pltpu.VMEM((NCH, CHUNK), jnp.int32),
            pltpu.VMEM((NCH, CHUNK), jnp.int32),
            pltpu.VMEM((2, K, CHUNK, d), jnp.float32),
            pltpu.VMEM_SHARED((NP, d), jnp.float32),
            pltpu.SemaphoreType.DMA,
            pltpu.SemaphoreType.DMA,
        ],
    )


def _make_sc_agg_spmem(d):
    mesh = plsc.VectorSubcoreMesh(core_axis_name="c", subcore_axis_name="s")

    def body(g_hbm, src_hbm, dst_hbm, zeros_hbm, out_hbm,
             src_v, dst_v, buf, table, acc, semg, sems):
        cid = lax.axis_index("c")
        sid = lax.axis_index("s")
        wid = cid * NS + sid
        pltpu.sync_copy(src_hbm.at[wid], src_v)
        pltpu.sync_copy(dst_hbm.at[wid], dst_v)
        sl = pl.ds(sid * RPS, RPS)
        pltpu.sync_copy(zeros_hbm.at[sl], acc.at[sl])
        pltpu.sync_copy(g_hbm.at[sl], table.at[sl])
        plsc.subcore_barrier()

        for j in range(K):
            pltpu.async_copy(table.at[src_v.at[j]], buf.at[0, j], semg)

        def group(i, carry):
            pb = lax.rem(i, 2)
            nb = 1 - pb
            base = i * K
            nbase = base + K

            @pl.when(i + 1 < NG)
            def _fire_next():
                for j in range(K):
                    pltpu.async_copy(table.at[src_v.at[nbase + j]],
                                     buf.at[nb, j], semg)

            for j in range(K):
                pltpu.make_async_copy(table.at[src_v.at[base + j]],
                                      buf.at[pb, j], semg).wait()
                pltpu.async_copy(buf.at[pb, j], acc.at[dst_v.at[base + j]],
                                 sems, add=True)
            for j in range(K):
                pltpu.make_async_copy(buf.at[pb, j],
                                      acc.at[dst_v.at[base + j]], sems).wait()
            return carry

        lax.fori_loop(0, NG, group, 0)
        plsc.subcore_barrier()
        pltpu.sync_copy(acc.at[sl], out_hbm.at[cid, sl])

    return pl.kernel(
        body,
        out_type=jax.ShapeDtypeStruct((NC, NP, d), jnp.float32),
        mesh=mesh,
        compiler_params=pltpu.CompilerParams(use_tc_tiling_on_sc=False),
        scratch_types=[
            pltpu.VMEM((NCH, CHUNK), jnp.int32),
            pltpu.VMEM((NCH, CHUNK), jnp.int32),
            pltpu.VMEM((2, K, CHUNK, d), jnp.float32),
            pltpu.VMEM_SHARED((NP, d), jnp.float32),
            pltpu.VMEM_SHARED((NP, d), jnp.float32),
            pltpu.SemaphoreType.DMA,
            pltpu.SemaphoreType.DMA,
        ],
    )


def _make_sc_deg():
    mesh = plsc.VectorSubcoreMesh(core_axis_name="c", subcore_axis_name="s")
    d = 8
    KD = 10

    def body(ones_hbm, dst_hbm, zeros_hbm, out_hbm, dst_v, ones_v, acc, sems):
        cid = lax.axis_index("c")
        sid = lax.axis_index("s")
        wid = cid * NS + sid
        pltpu.sync_copy(dst_hbm.at[wid], dst_v)
        pltpu.sync_copy(ones_hbm.at[pl.ds(0, CHUNK)], ones_v)
        sl = pl.ds(sid * RPS, RPS)
        pltpu.sync_copy(zeros_hbm.at[sl], acc.at[sl])
        plsc.subcore_barrier()

        def group(i, carry):
            base = i * KD
            for j in range(KD):
                pltpu.async_copy(ones_v, acc.at[dst_v.at[base + j]],
                                 sems, add=True)
            for j in range(KD):
                pltpu.make_async_copy(ones_v,
                                      acc.at[dst_v.at[base + j]], sems).wait()
            return carry

        lax.fori_loop(0, NCH // KD, group, 0)
        plsc.subcore_barrier()
        pltpu.sync_copy(acc.at[sl], out_hbm.at[cid, sl])

    return pl.kernel(
        body,
        out_type=jax.ShapeDtypeStruct((NC, NP, d), jnp.float32),
        mesh=mesh,
        compiler_params=pltpu.CompilerParams(use_tc_tiling_on_sc=False),
        scratch_types=[
            pltpu.VMEM((NCH, CHUNK), jnp.int32),
            pltpu.VMEM((CHUNK, d), jnp.float32),
            pltpu.VMEM_SHARED((NP, d), jnp.float32),
            pltpu.SemaphoreType.DMA,
        ],
    )


BM = 1024


def _tc_a_body(view_ref, w1t_ref, parts_ref, g1_ref, dis_ref):
    deg = parts_ref[0, :, 0:1] + parts_ref[1, :, 0:1] + 1.0
    dis = lax.rsqrt(deg)
    h1 = jnp.dot(view_ref[...], w1t_ref[...], preferred_element_type=jnp.float32)
    g1_ref[...] = h1 * dis
    dis_ref[...] = dis


def _tc_b_body(parts_ref, g1_ref, dis_ref, b1_ref, w2t_ref, g2_ref):
    dis = dis_ref[...]
    s = parts_ref[0] + parts_ref[1] + g1_ref[...]
    x = jnp.maximum(s * dis + b1_ref[...], 0.0)
    h2 = jnp.dot(x, w2t_ref[...], preferred_element_type=jnp.float32)
    g2_ref[...] = h2 * dis


def _tc_c_body(parts_ref, g2_ref, dis_ref, b2_ref, out_ref):
    s = parts_ref[0] + parts_ref[1] + g2_ref[...]
    out_ref[...] = s * dis_ref[...] + b2_ref[...]


def _row_spec(d):
    return pl.BlockSpec((BM, d), lambda i: (i, 0))


def _parts_spec(d):
    return pl.BlockSpec((NC, BM, d), lambda i: (0, i, 0))


def _full_spec(a, b):
    return pl.BlockSpec((a, b), lambda i: (0, 0))


_GRID = (NP // BM,)

_tc_a = pl.pallas_call(
    _tc_a_body,
    grid=_GRID,
    in_specs=[_row_spec(D_IN), _full_spec(D_IN, D_HID), _parts_spec(8)],
    out_specs=[_row_spec(D_HID), _row_spec(1)],
    out_shape=[
        jax.ShapeDtypeStruct((NP, D_HID), jnp.float32),
        jax.ShapeDtypeStruct((NP, 1), jnp.float32),
    ],
)

_tc_b = pl.pallas_call(
    _tc_b_body,
    grid=_GRID,
    in_specs=[_parts_spec(D_HID), _row_spec(D_HID), _row_spec(1),
              _full_spec(1, D_HID), _full_spec(D_HID, N_ACT)],
    out_specs=_row_spec(N_ACT),
    out_shape=jax.ShapeDtypeStruct((NP, N_ACT), jnp.float32),
)

_tc_c = pl.pallas_call(
    _tc_c_body,
    grid=_GRID,
    in_specs=[_parts_spec(N_ACT), _row_spec(N_ACT), _row_spec(1),
              _full_spec(1, N_ACT)],
    out_specs=_row_spec(N_ACT),
    out_shape=jax.ShapeDtypeStruct((NP, N_ACT), jnp.float32),
)

_agg_deg = _make_sc_deg()
_agg_h = _make_sc_agg(D_HID)
_agg_o = _make_sc_agg_spmem(N_ACT)


def kernel(view, edge_index, W1, b1, W2, b2):
    src = edge_index[0].astype(jnp.int32)
    dst = edge_index[1].astype(jnp.int32)
    pad = EP - N_EDGES
    fill = jnp.full((pad,), N_NODES, jnp.int32)
    src3 = jnp.concatenate([src, fill]).reshape(NW, NCH, CHUNK)
    dst3 = jnp.concatenate([dst, fill]).reshape(NW, NCH, CHUNK)

    view_p = jnp.pad(view, ((0, NP - N_NODES), (0, 0)))
    ones8 = jnp.ones((NP, 8), jnp.float32)
    z8 = jnp.zeros((NP, 8), jnp.float32)
    z64 = jnp.zeros((NP, D_HID), jnp.float32)
    z16 = jnp.zeros((NP, N_ACT), jnp.float32)

    deg_parts = _agg_deg(ones8, dst3, z8)
    g1, dis = _tc_a(view_p, W1.T, deg_parts)
    parts1 = _agg_h(g1, src3, dst3, z64)
    g2 = _tc_b(parts1, g1, dis, b1.reshape(1, D_HID), W2.T)
    parts2 = _agg_o(g2, src3, dst3, z16)
    out = _tc_c(parts2, g2, dis, b2.reshape(1, N_ACT))
    return out[:N_NODES]


# speedup vs baseline: 40.1042x; 1.4260x over previous
import functools

import jax
import jax.numpy as jnp
from jax import lax
from jax.experimental import pallas as pl
from jax.experimental.pallas import tpu as pltpu
from jax.experimental.pallas import tpu_sc as plsc

N_NODES = 10000
N_EDGES = 320000
D_IN = 128
D_HID = 64
N_ACT = 16

NP = 10240
NC, NS = 2, 16
NW = NC * NS
CHUNK = 128
K = 4
NG = 20
NCH = NG * K
EPW = NCH * CHUNK
EP = EPW * NW
RPS = NP // NS


def _make_sc_agg(d):
    mesh = plsc.VectorSubcoreMesh(core_axis_name="c", subcore_axis_name="s")

    def body(g_hbm, src_hbm, dst_hbm, zeros_hbm, out_hbm,
             src_v, dst_v, buf, acc, semg, sems):
        cid = lax.axis_index("c")
        sid = lax.axis_index("s")
        wid = cid * NS + sid
        pltpu.sync_copy(src_hbm.at[wid], src_v)
        pltpu.sync_copy(dst_hbm.at[wid], dst_v)
        sl = pl.ds(sid * RPS, RPS)
        pltpu.sync_copy(zeros_hbm.at[sl], acc.at[sl])
        plsc.subcore_barrier()

        for j in range(K):
            pltpu.async_copy(g_hbm.at[src_v.at[j]], buf.at[0, j], semg)

        def group(i, carry):
            pb = lax.rem(i, 2)
            nb = 1 - pb
            base = i * K
            nbase = base + K

            @pl.when(i + 1 < NG)
            def _fire_next():
                for j in range(K):
                    pltpu.async_copy(g_hbm.at[src_v.at[nbase + j]],
                                     buf.at[nb, j], semg)

            for j in range(K):
                pltpu.make_async_copy(g_hbm.at[src_v.at[base + j]],
                                      buf.at[pb, j], semg).wait()
                pltpu.async_copy(buf.at[pb, j], acc.at[dst_v.at[base + j]],
                                 sems, add=True)
            for j in range(K):
                pltpu.make_async_copy(buf.at[pb, j],
                                      acc.at[dst_v.at[base + j]], sems).wait()
            return carry

        lax.fori_loop(0, NG, group, 0)
        plsc.subcore_barrier()
        pltpu.sync_copy(acc.at[sl], out_hbm.at[cid, sl])

    return pl.kernel(
        body,
        out_type=jax.ShapeDtypeStruct((NC, NP, d), jnp.float32),
        mesh=mesh,
        compiler_params=pltpu.CompilerParams(use_tc_tiling_on_sc=False),
        scratch_types=[
            pltpu.VMEM((NCH, CHUNK), jnp.int32),
            pltpu.VMEM((NCH, CHUNK), jnp.int32),
            pltpu.VMEM((2, K, CHUNK, d), jnp.float32),
            pltpu.VMEM_SHARED((NP, d), jnp.float32),
            pltpu.SemaphoreType.DMA,
            pltpu.SemaphoreType.DMA,
        ],
    )


def _make_sc_agg_spmem(d):
    mesh = plsc.VectorSubcoreMesh(core_axis_name="c", subcore_axis_name="s")

    def body(g_hbm, src_hbm, dst_hbm, zeros_hbm, out_hbm,
             src_v, dst_v, buf, table, acc, semg, sems):
        cid = lax.axis_index("c")
        sid = lax.axis_index("s")
        wid = cid * NS + sid
        pltpu.sync_copy(src_hbm.at[wid], src_v)
        pltpu.sync_copy(dst_hbm.at[wid], dst_v)
        sl = pl.ds(sid * RPS, RPS)
        pltpu.sync_copy(zeros_hbm.at[sl], acc.at[sl])
        pltpu.sync_copy(g_hbm.at[sl], table.at[sl])
        plsc.subcore_barrier()

        for j in range(K):
            pltpu.async_copy(table.at[src_v.at[j]], buf.at[0, j], semg)

        def group(i, carry):
            pb = lax.rem(i, 2)
            nb = 1 - pb
            base = i * K
            nbase = base + K

            @pl.when(i + 1 < NG)
            def _fire_next():
                for j in range(K):
                    pltpu.async_copy(table.at[src_v.at[nbase + j]],
                                     buf.at[nb, j], semg)

            for j in range(K):
                pltpu.make_async_copy(table.at[src_v.at[base + j]],
                                      buf.at[pb, j], semg).wait()
                pltpu.async_copy(buf.at[pb, j], acc.at[dst_v.at[base + j]],
                                 sems, add=True)
            for j in range(K):
                pltpu.make_async_copy(buf.at[pb, j],
                                      acc.at[dst_v.at[base + j]], sems).wait()
            return carry

        lax.fori_loop(0, NG, group, 0)
        plsc.subcore_barrier()
        pltpu.sync_copy(acc.at[sl], out_hbm.at[cid, sl])

    return pl.kernel(
        body,
        out_type=jax.ShapeDtypeStruct((NC, NP, d), jnp.float32),
        mesh=mesh,
        compiler_params=pltpu.CompilerParams(use_tc_tiling_on_sc=False),
        scratch_types=[
            pltpu.VMEM((NCH, CHUNK), jnp.int32),
            pltpu.VMEM((NCH, CHUNK), jnp.int32),
            pltpu.VMEM((2, K, CHUNK, d), jnp.float32),
            pltpu.VMEM_SHARED((NP, d), jnp.float32),
            pltpu.VMEM_SHARED((NP, d), jnp.float32),
            pltpu.SemaphoreType.DMA,
            pltpu.SemaphoreType.DMA,
        ],
    )


def _make_sc_deg():
    mesh = plsc.VectorSubcoreMesh(core_axis_name="c", subcore_axis_name="s")
    d = 8
    KD = 10

    def body(ones_hbm, dst_hbm, zeros_hbm, out_hbm, dst_v, ones_v, acc, sems):
        cid = lax.axis_index("c")
        sid = lax.axis_index("s")
        wid = cid * NS + sid
        pltpu.sync_copy(dst_hbm.at[wid], dst_v)
        pltpu.sync_copy(ones_hbm.at[pl.ds(0, CHUNK)], ones_v)
        sl = pl.ds(sid * RPS, RPS)
        pltpu.sync_copy(zeros_hbm.at[sl], acc.at[sl])
        plsc.subcore_barrier()

        def group(i, carry):
            base = i * KD
            for j in range(KD):
                pltpu.async_copy(ones_v, acc.at[dst_v.at[base + j]],
                                 sems, add=True)
            for j in range(KD):
                pltpu.make_async_copy(ones_v,
                                      acc.at[dst_v.at[base + j]], sems).wait()
            return carry

        lax.fori_loop(0, NCH // KD, group, 0)
        plsc.subcore_barrier()
        pltpu.sync_copy(acc.at[sl], out_hbm.at[cid, sl])

    return pl.kernel(
        body,
        out_type=jax.ShapeDtypeStruct((NC, NP, d), jnp.float32),
        mesh=mesh,
        compiler_params=pltpu.CompilerParams(use_tc_tiling_on_sc=False),
        scratch_types=[
            pltpu.VMEM((NCH, CHUNK), jnp.int32),
            pltpu.VMEM((CHUNK, d), jnp.float32),
            pltpu.VMEM_SHARED((NP, d), jnp.float32),
            pltpu.SemaphoreType.DMA,
        ],
    )


BM = 1024


def _tc_a_body(view_ref, w1t_ref, parts_ref, g1a_ref, g1b_ref, dis_ref):
    deg = parts_ref[0, :, 0:1] + parts_ref[1, :, 0:1] + 1.0
    dis = lax.rsqrt(deg)
    h1 = jnp.dot(view_ref[...], w1t_ref[...], preferred_element_type=jnp.float32)
    g1 = h1 * dis
    g1a_ref[...] = g1[:, :D_HID // 2]
    g1b_ref[...] = g1[:, D_HID // 2:]
    dis_ref[...] = dis


def _tc_b_body(pa_ref, pb_ref, g1a_ref, g1b_ref, dis_ref, b1_ref, w2t_ref,
               g2_ref):
    dis = dis_ref[...]
    sa = pa_ref[0] + pa_ref[1] + g1a_ref[...]
    sb = pb_ref[0] + pb_ref[1] + g1b_ref[...]
    s = jnp.concatenate([sa, sb], axis=1)
    x = jnp.maximum(s * dis + b1_ref[...], 0.0)
    h2 = jnp.dot(x, w2t_ref[...], preferred_element_type=jnp.float32)
    g2_ref[...] = h2 * dis


def _tc_c_body(parts_ref, g2_ref, dis_ref, b2_ref, out_ref):
    s = parts_ref[0] + parts_ref[1] + g2_ref[...]
    out_ref[...] = s * dis_ref[...] + b2_ref[...]


def _row_spec(d):
    return pl.BlockSpec((BM, d), lambda i: (i, 0))


def _parts_spec(d):
    return pl.BlockSpec((NC, BM, d), lambda i: (0, i, 0))


def _full_spec(a, b):
    return pl.BlockSpec((a, b), lambda i: (0, 0))


_GRID = (NP // BM,)

_tc_a = pl.pallas_call(
    _tc_a_body,
    grid=_GRID,
    in_specs=[_row_spec(D_IN), _full_spec(D_IN, D_HID), _parts_spec(8)],
    out_specs=[_row_spec(D_HID // 2), _row_spec(D_HID // 2), _row_spec(1)],
    out_shape=[
        jax.ShapeDtypeStruct((NP, D_HID // 2), jnp.float32),
        jax.ShapeDtypeStruct((NP, D_HID // 2), jnp.float32),
        jax.ShapeDtypeStruct((NP, 1), jnp.float32),
    ],
)

_tc_b = pl.pallas_call(
    _tc_b_body,
    grid=_GRID,
    in_specs=[_parts_spec(D_HID // 2), _parts_spec(D_HID // 2),
              _row_spec(D_HID // 2), _row_spec(D_HID // 2), _row_spec(1),
              _full_spec(1, D_HID), _full_spec(D_HID, N_ACT)],
    out_specs=_row_spec(N_ACT),
    out_shape=jax.ShapeDtypeStruct((NP, N_ACT), jnp.float32),
)

_tc_c = pl.pallas_call(
    _tc_c_body,
    grid=_GRID,
    in_specs=[_parts_spec(N_ACT), _row_spec(N_ACT), _row_spec(1),
              _full_spec(1, N_ACT)],
    out_specs=_row_spec(N_ACT),
    out_shape=jax.ShapeDtypeStruct((NP, N_ACT), jnp.float32),
)

_agg_deg = _make_sc_deg()
_agg_h = _make_sc_agg_spmem(D_HID // 2)
_agg_o = _make_sc_agg_spmem(N_ACT)


def kernel(view, edge_index, W1, b1, W2, b2):
    src = edge_index[0].astype(jnp.int32)
    dst = edge_index[1].astype(jnp.int32)
    pad = EP - N_EDGES
    fill = jnp.full((pad,), N_NODES, jnp.int32)
    src3 = jnp.concatenate([src, fill]).reshape(NW, NCH, CHUNK)
    dst3 = jnp.concatenate([dst, fill]).reshape(NW, NCH, CHUNK)

    view_p = jnp.pad(view, ((0, NP - N_NODES), (0, 0)))
    ones8 = jnp.ones((NP, 8), jnp.float32)
    z8 = jnp.zeros((NP, 8), jnp.float32)
    z32 = jnp.zeros((NP, D_HID // 2), jnp.float32)
    z16 = jnp.zeros((NP, N_ACT), jnp.float32)

    deg_parts = _agg_deg(ones8, dst3, z8)
    g1a, g1b, dis = _tc_a(view_p, W1.T, deg_parts)
    parts1a = _agg_h(g1a, src3, dst3, z32)
    parts1b = _agg_h(g1b, src3, dst3, z32)
    g2 = _tc_b(parts1a, parts1b, g1a, g1b, dis,
               b1.reshape(1, D_HID), W2.T)
    parts2 = _agg_o(g2, src3, dst3, z16)
    out = _tc_c(parts2, g2, dis, b2.reshape(1, N_ACT))
    return out[:N_NODES]


# speedup vs baseline: 43.2371x; 1.0781x over previous
import functools

import jax
import jax.numpy as jnp
from jax import lax
from jax.experimental import pallas as pl
from jax.experimental.pallas import tpu as pltpu
from jax.experimental.pallas import tpu_sc as plsc

N_NODES = 10000
N_EDGES = 320000
D_IN = 128
D_HID = 64
N_ACT = 16

NP = 10240
NC, NS = 2, 16
NW = NC * NS
CHUNK = 128
K = 4
NG = 20
NCH = NG * K
EPW = NCH * CHUNK
EP = EPW * NW
RPS = NP // NS


def _make_sc_agg(d):
    mesh = plsc.VectorSubcoreMesh(core_axis_name="c", subcore_axis_name="s")

    def body(g_hbm, src_hbm, dst_hbm, zeros_hbm, out_hbm,
             src_v, dst_v, buf, acc, semg, sems):
        cid = lax.axis_index("c")
        sid = lax.axis_index("s")
        wid = cid * NS + sid
        pltpu.sync_copy(src_hbm.at[wid], src_v)
        pltpu.sync_copy(dst_hbm.at[wid], dst_v)
        sl = pl.ds(sid * RPS, RPS)
        pltpu.sync_copy(zeros_hbm.at[sl], acc.at[sl])
        plsc.subcore_barrier()

        for j in range(K):
            pltpu.async_copy(g_hbm.at[src_v.at[j]], buf.at[0, j], semg)

        def group(i, carry):
            pb = lax.rem(i, 2)
            nb = 1 - pb
            base = i * K
            nbase = base + K

            @pl.when(i + 1 < NG)
            def _fire_next():
                for j in range(K):
                    pltpu.async_copy(g_hbm.at[src_v.at[nbase + j]],
                                     buf.at[nb, j], semg)

            for j in range(K):
                pltpu.make_async_copy(g_hbm.at[src_v.at[base + j]],
                                      buf.at[pb, j], semg).wait()
                pltpu.async_copy(buf.at[pb, j], acc.at[dst_v.at[base + j]],
                                 sems, add=True)
            for j in range(K):
                pltpu.make_async_copy(buf.at[pb, j],
                                      acc.at[dst_v.at[base + j]], sems).wait()
            return carry

        lax.fori_loop(0, NG, group, 0)
        plsc.subcore_barrier()
        pltpu.sync_copy(acc.at[sl], out_hbm.at[cid, sl])

    return pl.kernel(
        body,
        out_type=jax.ShapeDtypeStruct((NC, NP, d), jnp.float32),
        mesh=mesh,
        compiler_params=pltpu.CompilerParams(use_tc_tiling_on_sc=False),
        scratch_types=[
            pltpu.VMEM((NCH, CHUNK), jnp.int32),
            pltpu.VMEM((NCH, CHUNK), jnp.int32),
            pltpu.VMEM((2, K, CHUNK, d), jnp.float32),
            pltpu.VMEM_SHARED((NP, d), jnp.float32),
            pltpu.SemaphoreType.DMA,
            pltpu.SemaphoreType.DMA,
        ],
    )


def _make_sc_agg_spmem(d):
    mesh = plsc.VectorSubcoreMesh(core_axis_name="c", subcore_axis_name="s")

    def body(g_hbm, src_hbm, dst_hbm, zeros_hbm, out_hbm,
             src_v, dst_v, buf, table, acc, semg, sems):
        cid = lax.axis_index("c")
        sid = lax.axis_index("s")
        wid = cid * NS + sid
        pltpu.sync_copy(src_hbm.at[wid], src_v)
        pltpu.sync_copy(dst_hbm.at[wid], dst_v)
        sl = pl.ds(sid * RPS, RPS)
        pltpu.sync_copy(zeros_hbm.at[sl], acc.at[sl])
        pltpu.sync_copy(g_hbm.at[sl], table.at[sl])
        plsc.subcore_barrier()

        for j in range(K):
            pltpu.async_copy(table.at[src_v.at[j]], buf.at[0, j], semg)

        def group(i, carry):
            pb = lax.rem(i, 2)
            nb = 1 - pb
            base = i * K
            nbase = base + K

            @pl.when(i + 1 < NG)
            def _fire_next():
                for j in range(K):
                    pltpu.async_copy(table.at[src_v.at[nbase + j]],
                                     buf.at[nb, j], semg)

            for j in range(K):
                pltpu.make_async_copy(table.at[src_v.at[base + j]],
                                      buf.at[pb, j], semg).wait()
                pltpu.async_copy(buf.at[pb, j], acc.at[dst_v.at[base + j]],
                                 sems, add=True)
            for j in range(K):
                pltpu.make_async_copy(buf.at[pb, j],
                                      acc.at[dst_v.at[base + j]], sems).wait()
            return carry

        lax.fori_loop(0, NG, group, 0)
        plsc.subcore_barrier()
        pltpu.sync_copy(acc.at[sl], out_hbm.at[cid, sl])

    return pl.kernel(
        body,
        out_type=jax.ShapeDtypeStruct((NC, NP, d), jnp.float32),
        mesh=mesh,
        compiler_params=pltpu.CompilerParams(use_tc_tiling_on_sc=False),
        scratch_types=[
            pltpu.VMEM((NCH, CHUNK), jnp.int32),
            pltpu.VMEM((NCH, CHUNK), jnp.int32),
            pltpu.VMEM((2, K, CHUNK, d), jnp.float32),
            pltpu.VMEM_SHARED((NP, d), jnp.float32),
            pltpu.VMEM_SHARED((NP, d), jnp.float32),
            pltpu.SemaphoreType.DMA,
            pltpu.SemaphoreType.DMA,
        ],
    )


NCH2 = EP // NS // CHUNK
NG2 = NCH2 // K


def _make_sc_agg_colsplit():
    mesh = plsc.VectorSubcoreMesh(core_axis_name="c", subcore_axis_name="s")
    d = D_HID // 2

    def body(g_hbm, src_hbm, dst_hbm, zeros_hbm, out_hbm,
             src_v, dst_v, buf, table, acc, semg, sems):
        cid = lax.axis_index("c")
        sid = lax.axis_index("s")
        pltpu.sync_copy(src_hbm.at[sid], src_v)
        pltpu.sync_copy(dst_hbm.at[sid], dst_v)
        sl = pl.ds(sid * RPS, RPS)
        cols = pl.ds(cid * d, d)
        pltpu.sync_copy(zeros_hbm.at[sl], acc.at[sl])
        pltpu.sync_copy(g_hbm.at[sl, cols], table.at[sl])
        plsc.subcore_barrier()

        for j in range(K):
            pltpu.async_copy(table.at[src_v.at[j]], buf.at[0, j], semg)

        def group(i, carry):
            pb = lax.rem(i, 2)
            nb = 1 - pb
            base = i * K
            nbase = base + K

            @pl.when(i + 1 < NG2)
            def _fire_next():
                for j in range(K):
                    pltpu.async_copy(table.at[src_v.at[nbase + j]],
                                     buf.at[nb, j], semg)

            for j in range(K):
                pltpu.make_async_copy(table.at[src_v.at[base + j]],
                                      buf.at[pb, j], semg).wait()
                pltpu.async_copy(buf.at[pb, j], acc.at[dst_v.at[base + j]],
                                 sems, add=True)
            for j in range(K):
                pltpu.make_async_copy(buf.at[pb, j],
                                      acc.at[dst_v.at[base + j]], sems).wait()
            return carry

        lax.fori_loop(0, NG2, group, 0)
        plsc.subcore_barrier()
        pltpu.sync_copy(acc.at[sl], out_hbm.at[sl, cols])

    return pl.kernel(
        body,
        out_type=jax.ShapeDtypeStruct((NP, D_HID), jnp.float32),
        mesh=mesh,
        compiler_params=pltpu.CompilerParams(use_tc_tiling_on_sc=False),
        scratch_types=[
            pltpu.VMEM((NCH2, CHUNK), jnp.int32),
            pltpu.VMEM((NCH2, CHUNK), jnp.int32),
            pltpu.VMEM((2, K, CHUNK, d), jnp.float32),
            pltpu.VMEM_SHARED((NP, d), jnp.float32),
            pltpu.VMEM_SHARED((NP, d), jnp.float32),
            pltpu.SemaphoreType.DMA,
            pltpu.SemaphoreType.DMA,
        ],
    )


def _make_sc_deg():
    mesh = plsc.VectorSubcoreMesh(core_axis_name="c", subcore_axis_name="s")
    d = 8
    KD = 10

    def body(ones_hbm, dst_hbm, zeros_hbm, out_hbm, dst_v, ones_v, acc, sems):
        cid = lax.axis_index("c")
        sid = lax.axis_index("s")
        wid = cid * NS + sid
        pltpu.sync_copy(dst_hbm.at[wid], dst_v)
        pltpu.sync_copy(ones_hbm.at[pl.ds(0, CHUNK)], ones_v)
        sl = pl.ds(sid * RPS, RPS)
        pltpu.sync_copy(zeros_hbm.at[sl], acc.at[sl])
        plsc.subcore_barrier()

        def group(i, carry):
            base = i * KD
            for j in range(KD):
                pltpu.async_copy(ones_v, acc.at[dst_v.at[base + j]],
                                 sems, add=True)
            for j in range(KD):
                pltpu.make_async_copy(ones_v,
                                      acc.at[dst_v.at[base + j]], sems).wait()
            return carry

        lax.fori_loop(0, NCH // KD, group, 0)
        plsc.subcore_barrier()
        pltpu.sync_copy(acc.at[sl], out_hbm.at[cid, sl])

    return pl.kernel(
        body,
        out_type=jax.ShapeDtypeStruct((NC, NP, d), jnp.float32),
        mesh=mesh,
        compiler_params=pltpu.CompilerParams(use_tc_tiling_on_sc=False),
        scratch_types=[
            pltpu.VMEM((NCH, CHUNK), jnp.int32),
            pltpu.VMEM((CHUNK, d), jnp.float32),
            pltpu.VMEM_SHARED((NP, d), jnp.float32),
            pltpu.SemaphoreType.DMA,
        ],
    )


BM = 1024


def _tc_a_body(view_ref, w1t_ref, parts_ref, g1_ref, dis_ref):
    deg = parts_ref[0, :, 0:1] + parts_ref[1, :, 0:1] + 1.0
    dis = lax.rsqrt(deg)
    h1 = jnp.dot(view_ref[...], w1t_ref[...], preferred_element_type=jnp.float32)
    g1_ref[...] = h1 * dis
    dis_ref[...] = dis


def _tc_b_body(p1_ref, g1_ref, dis_ref, b1_ref, w2t_ref, g2_ref):
    dis = dis_ref[...]
    s = p1_ref[...] + g1_ref[...]
    x = jnp.maximum(s * dis + b1_ref[...], 0.0)
    h2 = jnp.dot(x, w2t_ref[...], preferred_element_type=jnp.float32)
    g2_ref[...] = h2 * dis


def _tc_c_body(parts_ref, g2_ref, dis_ref, b2_ref, out_ref):
    s = parts_ref[0] + parts_ref[1] + g2_ref[...]
    out_ref[...] = s * dis_ref[...] + b2_ref[...]


def _row_spec(d):
    return pl.BlockSpec((BM, d), lambda i: (i, 0))


def _parts_spec(d):
    return pl.BlockSpec((NC, BM, d), lambda i: (0, i, 0))


def _full_spec(a, b):
    return pl.BlockSpec((a, b), lambda i: (0, 0))


_GRID = (NP // BM,)

_tc_a = pl.pallas_call(
    _tc_a_body,
    grid=_GRID,
    in_specs=[_row_spec(D_IN), _full_spec(D_IN, D_HID), _parts_spec(8)],
    out_specs=[_row_spec(D_HID), _row_spec(1)],
    out_shape=[
        jax.ShapeDtypeStruct((NP, D_HID), jnp.float32),
        jax.ShapeDtypeStruct((NP, 1), jnp.float32),
    ],
)

_tc_b = pl.pallas_call(
    _tc_b_body,
    grid=_GRID,
    in_specs=[_row_spec(D_HID), _row_spec(D_HID), _row_spec(1),
              _full_spec(1, D_HID), _full_spec(D_HID, N_ACT)],
    out_specs=_row_spec(N_ACT),
    out_shape=jax.ShapeDtypeStruct((NP, N_ACT), jnp.float32),
)

_tc_c = pl.pallas_call(
    _tc_c_body,
    grid=_GRID,
    in_specs=[_parts_spec(N_ACT), _row_spec(N_ACT), _row_spec(1),
              _full_spec(1, N_ACT)],
    out_specs=_row_spec(N_ACT),
    out_shape=jax.ShapeDtypeStruct((NP, N_ACT), jnp.float32),
)

_agg_deg = _make_sc_deg()
_agg_h = _make_sc_agg_colsplit()
_agg_o = _make_sc_agg_spmem(N_ACT)


def kernel(view, edge_index, W1, b1, W2, b2):
    src = edge_index[0].astype(jnp.int32)
    dst = edge_index[1].astype(jnp.int32)
    pad = EP - N_EDGES
    fill = jnp.full((pad,), N_NODES, jnp.int32)
    src3 = jnp.concatenate([src, fill]).reshape(NW, NCH, CHUNK)
    dst3 = jnp.concatenate([dst, fill]).reshape(NW, NCH, CHUNK)

    view_p = jnp.pad(view, ((0, NP - N_NODES), (0, 0)))
    src2 = src3.reshape(NS, NCH2, CHUNK)
    dst2 = dst3.reshape(NS, NCH2, CHUNK)

    ones8 = jnp.ones((NP, 8), jnp.float32)
    z8 = jnp.zeros((NP, 8), jnp.float32)
    z32 = jnp.zeros((NP, D_HID // 2), jnp.float32)
    z16 = jnp.zeros((NP, N_ACT), jnp.float32)

    deg_parts = _agg_deg(ones8, dst3, z8)
    g1, dis = _tc_a(view_p, W1.T, deg_parts)
    p1 = _agg_h(g1, src2, dst2, z32)
    g2 = _tc_b(p1, g1, dis, b1.reshape(1, D_HID), W2.T)
    parts2 = _agg_o(g2, src3, dst3, z16)
    out = _tc_c(parts2, g2, dis, b2.reshape(1, N_ACT))
    return out[:N_NODES]


# speedup vs baseline: 44.8043x; 1.0362x over previous
import functools

import jax
import jax.numpy as jnp
from jax import lax
from jax.experimental import pallas as pl
from jax.experimental.pallas import tpu as pltpu
from jax.experimental.pallas import tpu_sc as plsc

N_NODES = 10000
N_EDGES = 320000
D_IN = 128
D_HID = 64
N_ACT = 16

NP = 10240
NC, NS = 2, 16
NW = NC * NS
CHUNK = 128
K = 4
NG = 20
NCH = NG * K
EPW = NCH * CHUNK
EP = EPW * NW
RPS = NP // NS


def _make_sc_agg(d):
    mesh = plsc.VectorSubcoreMesh(core_axis_name="c", subcore_axis_name="s")

    def body(g_hbm, src_hbm, dst_hbm, zeros_hbm, out_hbm,
             src_v, dst_v, buf, acc, semg, sems):
        cid = lax.axis_index("c")
        sid = lax.axis_index("s")
        wid = cid * NS + sid
        pltpu.sync_copy(src_hbm.at[wid], src_v)
        pltpu.sync_copy(dst_hbm.at[wid], dst_v)
        sl = pl.ds(sid * RPS, RPS)
        pltpu.sync_copy(zeros_hbm.at[sl], acc.at[sl])
        plsc.subcore_barrier()

        for j in range(K):
            pltpu.async_copy(g_hbm.at[src_v.at[j]], buf.at[0, j], semg)

        def group(i, carry):
            pb = lax.rem(i, 2)
            nb = 1 - pb
            base = i * K
            nbase = base + K

            @pl.when(i + 1 < NG)
            def _fire_next():
                for j in range(K):
                    pltpu.async_copy(g_hbm.at[src_v.at[nbase + j]],
                                     buf.at[nb, j], semg)

            for j in range(K):
                pltpu.make_async_copy(g_hbm.at[src_v.at[base + j]],
                                      buf.at[pb, j], semg).wait()
                pltpu.async_copy(buf.at[pb, j], acc.at[dst_v.at[base + j]],
                                 sems, add=True)
            for j in range(K):
                pltpu.make_async_copy(buf.at[pb, j],
                                      acc.at[dst_v.at[base + j]], sems).wait()
            return carry

        lax.fori_loop(0, NG, group, 0)
        plsc.subcore_barrier()
        pltpu.sync_copy(acc.at[sl], out_hbm.at[cid, sl])

    return pl.kernel(
        body,
        out_type=jax.ShapeDtypeStruct((NC, NP, d), jnp.float32),
        mesh=mesh,
        compiler_params=pltpu.CompilerParams(use_tc_tiling_on_sc=False),
        scratch_types=[
            pltpu.VMEM((NCH, CHUNK), jnp.int32),
            pltpu.VMEM((NCH, CHUNK), jnp.int32),
            pltpu.VMEM((2, K, CHUNK, d), jnp.float32),
            pltpu.VMEM_SHARED((NP, d), jnp.float32),
            pltpu.SemaphoreType.DMA,
            pltpu.SemaphoreType.DMA,
        ],
    )


def _make_sc_agg_spmem(d):
    mesh = plsc.VectorSubcoreMesh(core_axis_name="c", subcore_axis_name="s")

    def body(g_hbm, src_hbm, dst_hbm, zeros_hbm, out_hbm,
             src_v, dst_v, buf, table, acc, semg, sems):
        cid = lax.axis_index("c")
        sid = lax.axis_index("s")
        wid = cid * NS + sid
        pltpu.sync_copy(src_hbm.at[wid], src_v)
        pltpu.sync_copy(dst_hbm.at[wid], dst_v)
        sl = pl.ds(sid * RPS, RPS)
        pltpu.sync_copy(zeros_hbm.at[sl], acc.at[sl])
        pltpu.sync_copy(g_hbm.at[sl], table.at[sl])
        plsc.subcore_barrier()

        for j in range(K):
            pltpu.async_copy(table.at[src_v.at[j]], buf.at[0, j], semg)

        def group(i, carry):
            pb = lax.rem(i, 2)
            nb = 1 - pb
            base = i * K
            nbase = base + K

            @pl.when(i + 1 < NG)
            def _fire_next():
                for j in range(K):
                    pltpu.async_copy(table.at[src_v.at[nbase + j]],
                                     buf.at[nb, j], semg)

            for j in range(K):
                pltpu.make_async_copy(table.at[src_v.at[base + j]],
                                      buf.at[pb, j], semg).wait()
                pltpu.async_copy(buf.at[pb, j], acc.at[dst_v.at[base + j]],
                                 sems, add=True)
            for j in range(K):
                pltpu.make_async_copy(buf.at[pb, j],
                                      acc.at[dst_v.at[base + j]], sems).wait()
            return carry

        lax.fori_loop(0, NG, group, 0)
        plsc.subcore_barrier()
        pltpu.sync_copy(acc.at[sl], out_hbm.at[cid, sl])

    return pl.kernel(
        body,
        out_type=jax.ShapeDtypeStruct((NC, NP, d), jnp.float32),
        mesh=mesh,
        compiler_params=pltpu.CompilerParams(use_tc_tiling_on_sc=False),
        scratch_types=[
            pltpu.VMEM((NCH, CHUNK), jnp.int32),
            pltpu.VMEM((NCH, CHUNK), jnp.int32),
            pltpu.VMEM((2, K, CHUNK, d), jnp.float32),
            pltpu.VMEM_SHARED((NP, d), jnp.float32),
            pltpu.VMEM_SHARED((NP, d), jnp.float32),
            pltpu.SemaphoreType.DMA,
            pltpu.SemaphoreType.DMA,
        ],
    )


NCH2 = EP // NS // CHUNK
NG2 = NCH2 // K


def _make_sc_agg_colsplit():
    mesh = plsc.VectorSubcoreMesh(core_axis_name="c", subcore_axis_name="s")
    d = D_HID // 2

    def body(g_hbm, src_hbm, dst_hbm, zeros_hbm, out_hbm,
             src_v, dst_v, buf, table, acc, semg, sems):
        cid = lax.axis_index("c")
        sid = lax.axis_index("s")
        pltpu.sync_copy(src_hbm.at[sid], src_v)
        pltpu.sync_copy(dst_hbm.at[sid], dst_v)
        sl = pl.ds(sid * RPS, RPS)
        cols = pl.ds(cid * d, d)
        pltpu.sync_copy(zeros_hbm.at[sl], acc.at[sl])
        pltpu.sync_copy(g_hbm.at[sl, cols], table.at[sl])
        plsc.subcore_barrier()

        for j in range(K):
            pltpu.async_copy(table.at[src_v.at[j]], buf.at[0, j], semg)

        def group(i, carry):
            pb = lax.rem(i, 2)
            nb = 1 - pb
            base = i * K
            nbase = base + K

            @pl.when(i + 1 < NG2)
            def _fire_next():
                for j in range(K):
                    pltpu.async_copy(table.at[src_v.at[nbase + j]],
                                     buf.at[nb, j], semg)

            for j in range(K):
                pltpu.make_async_copy(table.at[src_v.at[base + j]],
                                      buf.at[pb, j], semg).wait()
                pltpu.async_copy(buf.at[pb, j], acc.at[dst_v.at[base + j]],
                                 sems, add=True)
            for j in range(K):
                pltpu.make_async_copy(buf.at[pb, j],
                                      acc.at[dst_v.at[base + j]], sems).wait()
            return carry

        lax.fori_loop(0, NG2, group, 0)
        plsc.subcore_barrier()
        pltpu.sync_copy(acc.at[sl], out_hbm.at[sl, cols])

    return pl.kernel(
        body,
        out_type=jax.ShapeDtypeStruct((NP, D_HID), jnp.float32),
        mesh=mesh,
        compiler_params=pltpu.CompilerParams(use_tc_tiling_on_sc=False),
        scratch_types=[
            pltpu.VMEM((NCH2, CHUNK), jnp.int32),
            pltpu.VMEM((NCH2, CHUNK), jnp.int32),
            pltpu.VMEM((2, K, CHUNK, d), jnp.float32),
            pltpu.VMEM_SHARED((NP, d), jnp.float32),
            pltpu.VMEM_SHARED((NP, d), jnp.float32),
            pltpu.SemaphoreType.DMA,
            pltpu.SemaphoreType.DMA,
        ],
    )


def _make_sc_deg():
    mesh = plsc.VectorSubcoreMesh(core_axis_name="c", subcore_axis_name="s")
    d = 8
    KD = 10

    def body(ones_hbm, dst_hbm, zeros_hbm, out_hbm, dst_v, ones_v, acc, sems):
        cid = lax.axis_index("c")
        sid = lax.axis_index("s")
        wid = cid * NS + sid
        pltpu.sync_copy(dst_hbm.at[wid], dst_v)
        pltpu.sync_copy(ones_hbm.at[pl.ds(0, CHUNK)], ones_v)
        sl = pl.ds(sid * RPS, RPS)
        pltpu.sync_copy(zeros_hbm.at[sl], acc.at[sl])
        plsc.subcore_barrier()

        def group(i, carry):
            base = i * KD
            for j in range(KD):
                pltpu.async_copy(ones_v, acc.at[dst_v.at[base + j]],
                                 sems, add=True)
            for j in range(KD):
                pltpu.make_async_copy(ones_v,
                                      acc.at[dst_v.at[base + j]], sems).wait()
            return carry

        lax.fori_loop(0, NCH // KD, group, 0)
        plsc.subcore_barrier()
        pltpu.sync_copy(acc.at[sl], out_hbm.at[cid, sl])

    return pl.kernel(
        body,
        out_type=jax.ShapeDtypeStruct((NC, NP, d), jnp.float32),
        mesh=mesh,
        compiler_params=pltpu.CompilerParams(use_tc_tiling_on_sc=False),
        scratch_types=[
            pltpu.VMEM((NCH, CHUNK), jnp.int32),
            pltpu.VMEM((CHUNK, d), jnp.float32),
            pltpu.VMEM_SHARED((NP, d), jnp.float32),
            pltpu.SemaphoreType.DMA,
        ],
    )


BM = 2048


def _tc_a1_body(view_ref, w1t_ref, h1_ref):
    h1_ref[...] = jnp.dot(view_ref[...], w1t_ref[...],
                          preferred_element_type=jnp.float32)


def _tc_a2_body(h1_ref, parts_ref, g1_ref, dis_ref):
    deg = parts_ref[0, :, 0:1] + parts_ref[1, :, 0:1] + 1.0
    dis = lax.rsqrt(deg)
    g1_ref[...] = h1_ref[...] * dis
    dis_ref[...] = dis


def _tc_b_body(p1_ref, g1_ref, dis_ref, b1_ref, w2t_ref, g2_ref):
    dis = dis_ref[...]
    s = p1_ref[...] + g1_ref[...]
    x = jnp.maximum(s * dis + b1_ref[...], 0.0)
    h2 = jnp.dot(x, w2t_ref[...], preferred_element_type=jnp.float32)
    g2_ref[...] = h2 * dis


def _tc_c_body(parts_ref, g2_ref, dis_ref, b2_ref, out_ref):
    s = parts_ref[0] + parts_ref[1] + g2_ref[...]
    out_ref[...] = s * dis_ref[...] + b2_ref[...]


def _row_spec(d):
    return pl.BlockSpec((BM, d), lambda i: (i, 0))


def _parts_spec(d):
    return pl.BlockSpec((NC, BM, d), lambda i: (0, i, 0))


def _full_spec(a, b):
    return pl.BlockSpec((a, b), lambda i: (0, 0))


_GRID = (NP // BM,)

_tc_a1 = pl.pallas_call(
    _tc_a1_body,
    grid=_GRID,
    in_specs=[_row_spec(D_IN), _full_spec(D_IN, D_HID)],
    out_specs=_row_spec(D_HID),
    out_shape=jax.ShapeDtypeStruct((NP, D_HID), jnp.float32),
)

_tc_a2 = pl.pallas_call(
    _tc_a2_body,
    grid=_GRID,
    in_specs=[_row_spec(D_HID), _parts_spec(8)],
    out_specs=[_row_spec(D_HID), _row_spec(1)],
    out_shape=[
        jax.ShapeDtypeStruct((NP, D_HID), jnp.float32),
        jax.ShapeDtypeStruct((NP, 1), jnp.float32),
    ],
)

_tc_b = pl.pallas_call(
    _tc_b_body,
    grid=_GRID,
    in_specs=[_row_spec(D_HID), _row_spec(D_HID), _row_spec(1),
              _full_spec(1, D_HID), _full_spec(D_HID, N_ACT)],
    out_specs=_row_spec(N_ACT),
    out_shape=jax.ShapeDtypeStruct((NP, N_ACT), jnp.float32),
)

_tc_c = pl.pallas_call(
    _tc_c_body,
    grid=_GRID,
    in_specs=[_parts_spec(N_ACT), _row_spec(N_ACT), _row_spec(1),
              _full_spec(1, N_ACT)],
    out_specs=_row_spec(N_ACT),
    out_shape=jax.ShapeDtypeStruct((NP, N_ACT), jnp.float32),
)

_agg_deg = _make_sc_deg()
_agg_h = _make_sc_agg_colsplit()
_agg_o = _make_sc_agg_spmem(N_ACT)


def kernel(view, edge_index, W1, b1, W2, b2):
    src = edge_index[0].astype(jnp.int32)
    dst = edge_index[1].astype(jnp.int32)
    pad = EP - N_EDGES
    fill = jnp.full((pad,), N_NODES, jnp.int32)
    src3 = jnp.concatenate([src, fill]).reshape(NW, NCH, CHUNK)
    dst3 = jnp.concatenate([dst, fill]).reshape(NW, NCH, CHUNK)

    src2 = src3.reshape(NS, NCH2, CHUNK)
    dst2 = dst3.reshape(NS, NCH2, CHUNK)

    ones8 = jnp.ones((NP, 8), jnp.float32)
    z8 = jnp.zeros((NP, 8), jnp.float32)
    z32 = jnp.zeros((NP, D_HID // 2), jnp.float32)
    z16 = jnp.zeros((NP, N_ACT), jnp.float32)

    deg_parts = _agg_deg(ones8, dst3, z8)
    h1 = _tc_a1(view, W1.T)
    g1, dis = _tc_a2(h1, deg_parts)
    p1 = _agg_h(g1, src2, dst2, z32)
    g2 = _tc_b(p1, g1, dis, b1.reshape(1, D_HID), W2.T)
    parts2 = _agg_o(g2, src3, dst3, z16)
    out = _tc_c(parts2, g2, dis, b2.reshape(1, N_ACT))
    return out[:N_NODES]


# speedup vs baseline: 46.5951x; 1.0400x over previous
import functools

import jax
import jax.numpy as jnp
from jax import lax
from jax.experimental import pallas as pl
from jax.experimental.pallas import tpu as pltpu
from jax.experimental.pallas import tpu_sc as plsc

N_NODES = 10000
N_EDGES = 320000
D_IN = 128
D_HID = 64
N_ACT = 16

NP = 10240
NC, NS = 2, 16
NW = NC * NS
CHUNK = 128
K = 4
NG = 20
NCH = NG * K
EPW = NCH * CHUNK
EP = EPW * NW
RPS = NP // NS


def _make_sc_agg(d):
    mesh = plsc.VectorSubcoreMesh(core_axis_name="c", subcore_axis_name="s")

    def body(g_hbm, src_hbm, dst_hbm, zeros_hbm, out_hbm,
             src_v, dst_v, buf, acc, semg, sems):
        cid = lax.axis_index("c")
        sid = lax.axis_index("s")
        wid = cid * NS + sid
        pltpu.sync_copy(src_hbm.at[wid], src_v)
        pltpu.sync_copy(dst_hbm.at[wid], dst_v)
        sl = pl.ds(sid * RPS, RPS)
        pltpu.sync_copy(zeros_hbm.at[sl], acc.at[sl])
        plsc.subcore_barrier()

        for j in range(K):
            pltpu.async_copy(g_hbm.at[src_v.at[j]], buf.at[0, j], semg)

        def group(i, carry):
            pb = lax.rem(i, 2)
            nb = 1 - pb
            base = i * K
            nbase = base + K

            @pl.when(i + 1 < NG)
            def _fire_next():
                for j in range(K):
                    pltpu.async_copy(g_hbm.at[src_v.at[nbase + j]],
                                     buf.at[nb, j], semg)

            for j in range(K):
                pltpu.make_async_copy(g_hbm.at[src_v.at[base + j]],
                                      buf.at[pb, j], semg).wait()
                pltpu.async_copy(buf.at[pb, j], acc.at[dst_v.at[base + j]],
                                 sems, add=True)
            for j in range(K):
                pltpu.make_async_copy(buf.at[pb, j],
                                      acc.at[dst_v.at[base + j]], sems).wait()
            return carry

        lax.fori_loop(0, NG, group, 0)
        plsc.subcore_barrier()
        pltpu.sync_copy(acc.at[sl], out_hbm.at[cid, sl])

    return pl.kernel(
        body,
        out_type=jax.ShapeDtypeStruct((NC, NP, d), jnp.float32),
        mesh=mesh,
        compiler_params=pltpu.CompilerParams(use_tc_tiling_on_sc=False),
        scratch_types=[
            pltpu.VMEM((NCH, CHUNK), jnp.int32),
            pltpu.VMEM((NCH, CHUNK), jnp.int32),
            pltpu.VMEM((2, K, CHUNK, d), jnp.float32),
            pltpu.VMEM_SHARED((NP, d), jnp.float32),
            pltpu.SemaphoreType.DMA,
            pltpu.SemaphoreType.DMA,
        ],
    )


def _make_sc_agg_spmem(d, tc_tiling=False):
    mesh = plsc.VectorSubcoreMesh(core_axis_name="c", subcore_axis_name="s")

    def body(g_hbm, src_hbm, dst_hbm, zeros_hbm, out_hbm,
             src_v, dst_v, buf, table, acc, semg, sems):
        cid = lax.axis_index("c")
        sid = lax.axis_index("s")
        wid = cid * NS + sid
        pltpu.sync_copy(src_hbm.at[wid], src_v)
        pltpu.sync_copy(dst_hbm.at[wid], dst_v)
        sl = pl.ds(sid * RPS, RPS)
        pltpu.sync_copy(zeros_hbm.at[sl], acc.at[sl])
        pltpu.sync_copy(g_hbm.at[sl], table.at[sl])
        plsc.subcore_barrier()

        for j in range(K):
            pltpu.async_copy(table.at[src_v.at[j]], buf.at[0, j], semg)

        def group(i, carry):
            pb = lax.rem(i, 2)
            nb = 1 - pb
            base = i * K
            nbase = base + K

            @pl.when(i + 1 < NG)
            def _fire_next():
                for j in range(K):
                    pltpu.async_copy(table.at[src_v.at[nbase + j]],
                                     buf.at[nb, j], semg)

            for j in range(K):
                pltpu.make_async_copy(table.at[src_v.at[base + j]],
                                      buf.at[pb, j], semg).wait()
                pltpu.async_copy(buf.at[pb, j], acc.at[dst_v.at[base + j]],
                                 sems, add=True)
            for j in range(K):
                pltpu.make_async_copy(buf.at[pb, j],
                                      acc.at[dst_v.at[base + j]], sems).wait()
            return carry

        lax.fori_loop(0, NG, group, 0)
        plsc.subcore_barrier()
        pltpu.sync_copy(acc.at[sl], out_hbm.at[cid, sl])

    return pl.kernel(
        body,
        out_type=jax.ShapeDtypeStruct((NC, NP, d), jnp.float32),
        mesh=mesh,
        compiler_params=pltpu.CompilerParams(use_tc_tiling_on_sc=tc_tiling),
        scratch_types=[
            pltpu.VMEM((NCH, CHUNK), jnp.int32),
            pltpu.VMEM((NCH, CHUNK), jnp.int32),
            pltpu.VMEM((2, K, CHUNK, d), jnp.float32),
            pltpu.VMEM_SHARED((NP, d), jnp.float32),
            pltpu.VMEM_SHARED((NP, d), jnp.float32),
            pltpu.SemaphoreType.DMA,
            pltpu.SemaphoreType.DMA,
        ],
    )


NCH2 = EP // NS // CHUNK
NG2 = NCH2 // K


def _make_sc_agg_colsplit():
    mesh = plsc.VectorSubcoreMesh(core_axis_name="c", subcore_axis_name="s")
    d = D_HID // 2

    def body(g_hbm, src_hbm, dst_hbm, zeros_hbm, out_hbm,
             src_v, dst_v, buf, table, acc, semg, sems):
        cid = lax.axis_index("c")
        sid = lax.axis_index("s")
        pltpu.sync_copy(src_hbm.at[sid], src_v)
        pltpu.sync_copy(dst_hbm.at[sid], dst_v)
        sl = pl.ds(sid * RPS, RPS)
        cols = pl.ds(cid * d, d)
        pltpu.sync_copy(zeros_hbm.at[sl], acc.at[sl])
        pltpu.sync_copy(g_hbm.at[sl, cols], table.at[sl])
        plsc.subcore_barrier()

        for j in range(K):
            pltpu.async_copy(table.at[src_v.at[j]], buf.at[0, j], semg)

        def group(i, carry):
            pb = lax.rem(i, 2)
            nb = 1 - pb
            base = i * K
            nbase = base + K

            @pl.when(i + 1 < NG2)
            def _fire_next():
                for j in range(K):
                    pltpu.async_copy(table.at[src_v.at[nbase + j]],
                                     buf.at[nb, j], semg)

            for j in range(K):
                pltpu.make_async_copy(table.at[src_v.at[base + j]],
                                      buf.at[pb, j], semg).wait()
                pltpu.async_copy(buf.at[pb, j], acc.at[dst_v.at[base + j]],
                                 sems, add=True)
            for j in range(K):
                pltpu.make_async_copy(buf.at[pb, j],
                                      acc.at[dst_v.at[base + j]], sems).wait()
            return carry

        lax.fori_loop(0, NG2, group, 0)
        plsc.subcore_barrier()
        pltpu.sync_copy(acc.at[sl], out_hbm.at[sl, cols])

    return pl.kernel(
        body,
        out_type=jax.ShapeDtypeStruct((NP, D_HID), jnp.float32),
        mesh=mesh,
        compiler_params=pltpu.CompilerParams(use_tc_tiling_on_sc=False),
        scratch_types=[
            pltpu.VMEM((NCH2, CHUNK), jnp.int32),
            pltpu.VMEM((NCH2, CHUNK), jnp.int32),
            pltpu.VMEM((2, K, CHUNK, d), jnp.float32),
            pltpu.VMEM_SHARED((NP, d), jnp.float32),
            pltpu.VMEM_SHARED((NP, d), jnp.float32),
            pltpu.SemaphoreType.DMA,
            pltpu.SemaphoreType.DMA,
        ],
    )


def _make_sc_deg():
    mesh = plsc.VectorSubcoreMesh(core_axis_name="c", subcore_axis_name="s")
    d = 8
    KD = 10

    def body(ones_hbm, dst_hbm, zeros_hbm, out_hbm, dst_v, ones_v, acc, sems):
        cid = lax.axis_index("c")
        sid = lax.axis_index("s")
        wid = cid * NS + sid
        pltpu.sync_copy(dst_hbm.at[wid], dst_v)
        pltpu.sync_copy(ones_hbm.at[pl.ds(0, CHUNK)], ones_v)
        sl = pl.ds(sid * RPS, RPS)
        pltpu.sync_copy(zeros_hbm.at[sl], acc.at[sl])
        plsc.subcore_barrier()

        def group(i, carry):
            base = i * KD
            for j in range(KD):
                pltpu.async_copy(ones_v, acc.at[dst_v.at[base + j]],
                                 sems, add=True)
            for j in range(KD):
                pltpu.make_async_copy(ones_v,
                                      acc.at[dst_v.at[base + j]], sems).wait()
            return carry

        lax.fori_loop(0, NCH // KD, group, 0)
        plsc.subcore_barrier()
        pltpu.sync_copy(acc.at[sl], out_hbm.at[cid, sl])

    return pl.kernel(
        body,
        out_type=jax.ShapeDtypeStruct((NC, NP, 8), jnp.float32),
        mesh=mesh,
        compiler_params=pltpu.CompilerParams(use_tc_tiling_on_sc=False),
        scratch_types=[
            pltpu.VMEM((NCH, CHUNK), jnp.int32),
            pltpu.VMEM((CHUNK, d), jnp.float32),
            pltpu.VMEM_SHARED((NP, d), jnp.float32),
            pltpu.SemaphoreType.DMA,
        ],
    )


BM = 2048


def _tc_a1_body(view_ref, w1t_ref, h1_ref):
    h1_ref[...] = jnp.dot(view_ref[...], w1t_ref[...],
                          preferred_element_type=jnp.float32)


def _tc_a2_body(h1_ref, parts_ref, g1_ref, dis_ref):
    deg = parts_ref[0, :, 0:1] + parts_ref[1, :, 0:1] + 1.0
    dis = lax.rsqrt(deg)
    g1_ref[...] = h1_ref[...] * dis
    dis_ref[...] = dis


def _tc_b_body(p1_ref, g1_ref, dis_ref, b1_ref, w2t_ref, g2_ref):
    dis = dis_ref[...]
    s = p1_ref[...] + g1_ref[...]
    x = jnp.maximum(s * dis + b1_ref[...], 0.0)
    h2 = jnp.dot(x, w2t_ref[...], preferred_element_type=jnp.float32)
    g2_ref[...] = h2 * dis


def _tc_c_body(parts_ref, g2_ref, dis_ref, b2_ref, out_ref):
    s = parts_ref[0] + parts_ref[1] + g2_ref[...]
    out_ref[...] = s * dis_ref[...] + b2_ref[...]


def _row_spec(d):
    return pl.BlockSpec((BM, d), lambda i: (i, 0))


def _parts_spec(d):
    return pl.BlockSpec((NC, BM, d), lambda i: (0, i, 0))


def _full_spec(a, b):
    return pl.BlockSpec((a, b), lambda i: (0, 0))


_GRID = (NP // BM,)

_tc_a1 = pl.pallas_call(
    _tc_a1_body,
    grid=_GRID,
    in_specs=[_row_spec(D_IN), _full_spec(D_IN, D_HID)],
    out_specs=_row_spec(D_HID),
    out_shape=jax.ShapeDtypeStruct((NP, D_HID), jnp.float32),
)

_tc_a2 = pl.pallas_call(
    _tc_a2_body,
    grid=_GRID,
    in_specs=[_row_spec(D_HID), _parts_spec(8)],
    out_specs=[_row_spec(D_HID), _row_spec(1)],
    out_shape=[
        jax.ShapeDtypeStruct((NP, D_HID), jnp.float32),
        jax.ShapeDtypeStruct((NP, 1), jnp.float32),
    ],
)

_tc_b = pl.pallas_call(
    _tc_b_body,
    grid=_GRID,
    in_specs=[_row_spec(D_HID), _row_spec(D_HID), _row_spec(1),
              _full_spec(1, D_HID), _full_spec(D_HID, N_ACT)],
    out_specs=_row_spec(N_ACT),
    out_shape=jax.ShapeDtypeStruct((NP, N_ACT), jnp.float32),
)

_tc_c = pl.pallas_call(
    _tc_c_body,
    grid=_GRID,
    in_specs=[_parts_spec(N_ACT), _row_spec(N_ACT), _row_spec(1),
              _full_spec(1, N_ACT)],
    out_specs=_row_spec(N_ACT),
    out_shape=jax.ShapeDtypeStruct((NP, N_ACT), jnp.float32),
)

_agg_deg = _make_sc_deg()
_agg_h = _make_sc_agg_colsplit()
_agg_o = _make_sc_agg_spmem(N_ACT)


def kernel(view, edge_index, W1, b1, W2, b2):
    ei = jnp.pad(edge_index.astype(jnp.int32), ((0, 0), (0, EP - N_EDGES)),
                 constant_values=N_NODES)
    src3 = ei[0].reshape(NW, NCH, CHUNK)
    dst3 = ei[1].reshape(NW, NCH, CHUNK)
    src2 = ei[0].reshape(NS, NCH2, CHUNK)
    dst2 = ei[1].reshape(NS, NCH2, CHUNK)

    ones8 = jnp.ones((NP, 8), jnp.float32)
    z8 = jnp.zeros((NP, 8), jnp.float32)
    z32 = jnp.zeros((NP, D_HID // 2), jnp.float32)
    z16 = jnp.zeros((NP, N_ACT), jnp.float32)

    deg_parts = _agg_deg(ones8, dst3, z8)
    h1 = _tc_a1(view, W1.T)
    g1, dis = _tc_a2(h1, deg_parts)
    p1 = _agg_h(g1, src2, dst2, z32)
    g2 = _tc_b(p1, g1, dis, b1.reshape(1, D_HID), W2.T)
    parts2 = _agg_o(g2, src3, dst3, z16)
    out = _tc_c(parts2, g2, dis, b2.reshape(1, N_ACT))
    return out[:N_NODES]
